# baseline (speedup 1.0000x reference)
import functools
import math

import jax
import jax.numpy as jnp
from jax import lax
from jax.experimental import pallas as pl
from jax.experimental.pallas import tpu as pltpu
from jax.experimental.pallas import tpu_sc as plsc

T = 2048
M = 1024
E = 3
EP = 8
CAP = 683
CPAD = 688
EC = E * CPAD
DUMP = CAP
TBLK = 128
NBLK = T // TBLK


def _gating_body(x_ref, wg_ref, gm_ref, idxc_ref, idxd_ref, cnt_ref, laux_ref,
                 offs, me_acc, ce_acc):
    i = pl.program_id(0)

    @pl.when(i == 0)
    def _init():
        offs[...] = jnp.zeros_like(offs)
        me_acc[...] = jnp.zeros_like(me_acc)
        ce_acc[...] = jnp.zeros_like(ce_acc)

    xb = x_ref[...]
    logits = jnp.dot(xb, wg_ref[...], preferred_element_type=jnp.float32)
    col = lax.broadcasted_iota(jnp.int32, (TBLK, EP), 1)
    valid = col < E
    neg = jnp.float32(-1e30)
    logits = jnp.where(valid, logits, neg)

    mx = jnp.max(logits, axis=1, keepdims=True)
    ex = jnp.exp(logits - mx)
    ex = jnp.where(valid, ex, 0.0)
    gates = ex / jnp.sum(ex, axis=1, keepdims=True)

    gmax = jnp.max(gates, axis=1, keepdims=True)
    iseq = jnp.logical_and(gates == gmax, valid)
    e_s = jnp.min(jnp.where(iseq, col, 999), axis=1, keepdims=True)
    mask1 = jnp.where(col == e_s, 1.0, 0.0)

    ri = lax.broadcasted_iota(jnp.int32, (TBLK, TBLK), 0)
    ci = lax.broadcasted_iota(jnp.int32, (TBLK, TBLK), 1)
    lt = jnp.where(ci < ri, 1.0, 0.0).astype(jnp.float32)
    pos = jnp.dot(lt, mask1, preferred_element_type=jnp.float32)
    loc = pos + offs[...]

    keep = mask1 * jnp.where(loc < CAP, 1.0, 0.0)
    gm = jnp.sum(gates * keep, axis=1, keepdims=True)
    c_s = jnp.sum(loc * keep, axis=1, keepdims=True)
    kept = jnp.sum(keep, axis=1, keepdims=True)
    slot = e_s.astype(jnp.float32) * CPAD + c_s
    gm_ref[...] = gm
    idxc_ref[...] = (kept * slot).astype(jnp.int32)
    idxd_ref[...] = (kept * slot + (1.0 - kept) * DUMP).astype(jnp.int32)

    new_offs = offs[...] + jnp.sum(mask1, axis=0, keepdims=True)
    offs[...] = new_offs
    me_acc[...] = me_acc[...] + jnp.sum(gates, axis=0, keepdims=True)
    ce_acc[...] = ce_acc[...] + jnp.sum(mask1, axis=0, keepdims=True)

    @pl.when(i == NBLK - 1)
    def _fini():
        cnt_ref[...] = jnp.minimum(new_offs, float(CAP)).astype(jnp.int32)
        me = me_acc[...] / T
        ce = ce_acc[...] / T
        laux_ref[...] = jnp.sum(me * ce, axis=1, keepdims=True) * E


_gating_in_specs = [
    pl.BlockSpec((TBLK, M), lambda i: (i, 0)),
    pl.BlockSpec((M, EP), lambda i: (0, 0)),
]
_gating_out_specs = [
    pl.BlockSpec((TBLK, 1), lambda i: (i, 0)),
    pl.BlockSpec((TBLK, 1), lambda i: (i, 0)),
    pl.BlockSpec((TBLK, 1), lambda i: (i, 0)),
    pl.BlockSpec((1, EP), lambda i: (0, 0)),
    pl.BlockSpec((1, 1), lambda i: (0, 0)),
]
_gating_out_shape = [
    jax.ShapeDtypeStruct((T, 1), jnp.float32),
    jax.ShapeDtypeStruct((T, 1), jnp.int32),
    jax.ShapeDtypeStruct((T, 1), jnp.int32),
    jax.ShapeDtypeStruct((1, EP), jnp.int32),
    jax.ShapeDtypeStruct((1, 1), jnp.float32),
]
_gating_scratch = [
    pltpu.VMEM((1, EP), jnp.float32),
    pltpu.VMEM((1, EP), jnp.float32),
    pltpu.VMEM((1, EP), jnp.float32),
]

_gating = pl.pallas_call(
    _gating_body,
    grid=(NBLK,),
    in_specs=_gating_in_specs,
    out_specs=_gating_out_specs,
    out_shape=_gating_out_shape,
    scratch_shapes=_gating_scratch,
)


_NC = 2
_NS = 16
_NW = _NC * _NS
_ROWS_PER_W = T // _NW

def _dispatch_body(x_hbm, idx_hbm, out_hbm, idx_v, rows_v, sem):
    wid = lax.axis_index("s") * _NC + lax.axis_index("c")
    base = wid * _ROWS_PER_W
    pltpu.sync_copy(idx_hbm.at[pl.ds(base, _ROWS_PER_W)], idx_v)
    pltpu.sync_copy(x_hbm.at[pl.ds(base, _ROWS_PER_W)], rows_v)
    pltpu.async_copy(rows_v, out_hbm.at[idx_v], sem).wait()


def _combine_body(eo_hbm, idx_hbm, out_hbm, idx_v, rows_v, sem):
    wid = lax.axis_index("s") * _NC + lax.axis_index("c")
    base = wid * _ROWS_PER_W
    pltpu.sync_copy(idx_hbm.at[pl.ds(base, _ROWS_PER_W)], idx_v)
    pltpu.async_copy(eo_hbm.at[idx_v], rows_v, sem).wait()
    pltpu.sync_copy(rows_v, out_hbm.at[pl.ds(base, _ROWS_PER_W)])


@functools.lru_cache(maxsize=None)
def _sc_kernels():
    mesh = plsc.VectorSubcoreMesh(
        core_axis_name="c", subcore_axis_name="s",
        num_cores=_NC, num_subcores=_NS,
    )
    scratch = [
        pltpu.VMEM((_ROWS_PER_W,), jnp.int32),
        pltpu.VMEM((_ROWS_PER_W, M), jnp.float32),
        pltpu.SemaphoreType.DMA,
    ]
    dispatch = pl.kernel(
        _dispatch_body,
        out_type=jax.ShapeDtypeStruct((EC, M), jnp.float32),
        mesh=mesh,
        scratch_types=scratch,
    )
    combine = pl.kernel(
        _combine_body,
        out_type=jax.ShapeDtypeStruct((T, M), jnp.float32),
        mesh=mesh,
        scratch_types=scratch,
    )
    return dispatch, combine


def _expert_body(cnt_ref, disp_ref, w_ref, b_ref, out_ref):
    e = pl.program_id(0)
    cnt = cnt_ref[0, e]
    ri = lax.broadcasted_iota(jnp.int32, (CPAD, M), 0)
    xb = jnp.where(ri < cnt, disp_ref[...], 0.0)
    out_ref[...] = (
        jnp.dot(xb, w_ref[0], preferred_element_type=jnp.float32) + b_ref[0]
    )


_expert_in_specs = [
    pl.BlockSpec(memory_space=pltpu.SMEM),
    pl.BlockSpec((CPAD, M), lambda e: (e, 0)),
    pl.BlockSpec((1, M, M), lambda e: (e, 0, 0)),
    pl.BlockSpec((1, 1, M), lambda e: (e, 0, 0)),
]
_expert_out_specs = pl.BlockSpec((CPAD, M), lambda e: (e, 0))

_expert_mm = pl.pallas_call(
    _expert_body,
    grid=(E,),
    in_specs=_expert_in_specs,
    out_specs=_expert_out_specs,
    out_shape=jax.ShapeDtypeStruct((EC, M), jnp.float32),
)


_SBLK = 512


def _scale_body(y_ref, gm_ref, out_ref):
    out_ref[...] = y_ref[...] * gm_ref[...]


_scale_in_specs = [
    pl.BlockSpec((_SBLK, M), lambda i: (i, 0)),
    pl.BlockSpec((_SBLK, 1), lambda i: (i, 0)),
]
_scale_out_specs = pl.BlockSpec((_SBLK, M), lambda i: (i, 0))

_scale = pl.pallas_call(
    _scale_body,
    grid=(T // _SBLK,),
    in_specs=_scale_in_specs,
    out_specs=_scale_out_specs,
    out_shape=jax.ShapeDtypeStruct((T, M), jnp.float32),
)


def kernel(features, wg, W, b):
    B, S, _ = features.shape
    x = features.reshape(T, M)
    wg8 = jnp.pad(wg, ((0, 0), (0, EP - E)))
    dispatch, combine = _sc_kernels()
    gm, idx_c, idx_d, counts, laux = _gating(x, wg8)
    disp = dispatch(x, idx_d.reshape(T))
    eo = _expert_mm(counts, disp, W, b.reshape(E, 1, M))
    comb = combine(eo, idx_c.reshape(T))
    out = _scale(comb, gm)
    return out.reshape(B, S, M), laux[0, 0]


# speedup vs baseline: 1.0882x; 1.0882x over previous
import functools
import math

import jax
import jax.numpy as jnp
from jax import lax
from jax.experimental import pallas as pl
from jax.experimental.pallas import tpu as pltpu
from jax.experimental.pallas import tpu_sc as plsc

T = 2048
M = 1024
E = 3
EP = 8
CAP = 683
CPAD = 688
EC = E * CPAD
DUMP = CAP
GL = 128
TBLK = 128
NBLK = T // TBLK


def _gating_body(x_ref, wg_ref, gm_ref, idx_ref, cnt_ref, laux_ref,
                 offs, me_acc, ce_acc):
    i = pl.program_id(0)

    @pl.when(i == 0)
    def _init():
        offs[...] = jnp.zeros_like(offs)
        me_acc[...] = jnp.zeros_like(me_acc)
        ce_acc[...] = jnp.zeros_like(ce_acc)

    xb = x_ref[...]
    logits = jnp.dot(xb, wg_ref[...], preferred_element_type=jnp.float32)
    col = lax.broadcasted_iota(jnp.int32, (TBLK, EP), 1)
    valid = col < E
    neg = jnp.float32(-1e30)
    logits = jnp.where(valid, logits, neg)

    mx = jnp.max(logits, axis=1, keepdims=True)
    ex = jnp.exp(logits - mx)
    ex = jnp.where(valid, ex, 0.0)
    gates = ex / jnp.sum(ex, axis=1, keepdims=True)

    gmax = jnp.max(gates, axis=1, keepdims=True)
    iseq = jnp.logical_and(gates == gmax, valid)
    e_s = jnp.min(jnp.where(iseq, col, 999), axis=1, keepdims=True)
    mask1 = jnp.where(col == e_s, 1.0, 0.0)

    ri = lax.broadcasted_iota(jnp.int32, (TBLK, TBLK), 0)
    ci = lax.broadcasted_iota(jnp.int32, (TBLK, TBLK), 1)
    lt = jnp.where(ci < ri, 1.0, 0.0).astype(jnp.float32)
    pos = jnp.dot(lt, mask1, preferred_element_type=jnp.float32)
    loc = pos + offs[...]

    keep = mask1 * jnp.where(loc < CAP, 1.0, 0.0)
    gm = jnp.sum(gates * keep, axis=1, keepdims=True)
    c_s = jnp.sum(loc * keep, axis=1, keepdims=True)
    kept = jnp.sum(keep, axis=1, keepdims=True)
    slot = e_s.astype(jnp.float32) * CPAD + c_s
    gm_ref[...] = jnp.broadcast_to(gm, (TBLK, GL))
    idx_ref[...] = (kept * slot + (1.0 - kept) * DUMP).astype(jnp.int32)

    new_offs = offs[...] + jnp.sum(mask1, axis=0, keepdims=True)
    offs[...] = new_offs
    me_acc[...] = me_acc[...] + jnp.sum(gates, axis=0, keepdims=True)
    ce_acc[...] = ce_acc[...] + jnp.sum(mask1, axis=0, keepdims=True)

    @pl.when(i == NBLK - 1)
    def _fini():
        cnt_ref[...] = jnp.minimum(new_offs, float(CAP)).astype(jnp.int32)
        me = me_acc[...] / T
        ce = ce_acc[...] / T
        laux_ref[...] = jnp.sum(me * ce, axis=1, keepdims=True) * E


_gating_in_specs = [
    pl.BlockSpec((TBLK, M), lambda i: (i, 0)),
    pl.BlockSpec((M, EP), lambda i: (0, 0)),
]
_gating_out_specs = [
    pl.BlockSpec((TBLK, GL), lambda i: (i, 0)),
    pl.BlockSpec((TBLK, 1), lambda i: (i, 0)),
    pl.BlockSpec((1, EP), lambda i: (0, 0)),
    pl.BlockSpec((1, 1), lambda i: (0, 0)),
]
_gating_out_shape = [
    jax.ShapeDtypeStruct((T, GL), jnp.float32),
    jax.ShapeDtypeStruct((T, 1), jnp.int32),
    jax.ShapeDtypeStruct((1, EP), jnp.int32),
    jax.ShapeDtypeStruct((1, 1), jnp.float32),
]
_gating_scratch = [
    pltpu.VMEM((1, EP), jnp.float32),
    pltpu.VMEM((1, EP), jnp.float32),
    pltpu.VMEM((1, EP), jnp.float32),
]

_gating = pl.pallas_call(
    _gating_body,
    grid=(NBLK,),
    in_specs=_gating_in_specs,
    out_specs=_gating_out_specs,
    out_shape=_gating_out_shape,
    scratch_shapes=_gating_scratch,
)


_NC = 2
_NS = 16
_NW = _NC * _NS
_ROWS_PER_W = T // _NW


def _dispatch_body(x_hbm, gm_hbm, idx_hbm, disp_hbm, gslot_hbm,
                   idx_v, rows_v, gm_v, sem, sem2):
    wid = lax.axis_index("s") * _NC + lax.axis_index("c")
    base = wid * _ROWS_PER_W
    pltpu.sync_copy(idx_hbm.at[pl.ds(base, _ROWS_PER_W)], idx_v)
    pltpu.sync_copy(x_hbm.at[pl.ds(base, _ROWS_PER_W)], rows_v)
    pltpu.sync_copy(gm_hbm.at[pl.ds(base, _ROWS_PER_W)], gm_v)
    row_cp = pltpu.async_copy(rows_v, disp_hbm.at[idx_v], sem)
    gm_cp = pltpu.async_copy(gm_v, gslot_hbm.at[idx_v], sem2)
    row_cp.wait()
    gm_cp.wait()


def _combine_body(eo_hbm, idx_hbm, out_hbm, idx_v, rows_v, sem):
    wid = lax.axis_index("s") * _NC + lax.axis_index("c")
    base = wid * _ROWS_PER_W
    pltpu.sync_copy(idx_hbm.at[pl.ds(base, _ROWS_PER_W)], idx_v)
    pltpu.async_copy(eo_hbm.at[idx_v], rows_v, sem).wait()
    pltpu.sync_copy(rows_v, out_hbm.at[pl.ds(base, _ROWS_PER_W)])


@functools.lru_cache(maxsize=None)
def _sc_kernels():
    mesh = plsc.VectorSubcoreMesh(
        core_axis_name="c", subcore_axis_name="s",
        num_cores=_NC, num_subcores=_NS,
    )
    dispatch = pl.kernel(
        _dispatch_body,
        out_type=[
            jax.ShapeDtypeStruct((EC, M), jnp.float32),
            jax.ShapeDtypeStruct((EC, GL), jnp.float32),
        ],
        mesh=mesh,
        scratch_types=[
            pltpu.VMEM((_ROWS_PER_W,), jnp.int32),
            pltpu.VMEM((_ROWS_PER_W, M), jnp.float32),
            pltpu.VMEM((_ROWS_PER_W, GL), jnp.float32),
            pltpu.SemaphoreType.DMA,
            pltpu.SemaphoreType.DMA,
        ],
    )
    combine = pl.kernel(
        _combine_body,
        out_type=jax.ShapeDtypeStruct((T, M), jnp.float32),
        mesh=mesh,
        scratch_types=[
            pltpu.VMEM((_ROWS_PER_W,), jnp.int32),
            pltpu.VMEM((_ROWS_PER_W, M), jnp.float32),
            pltpu.SemaphoreType.DMA,
        ],
    )
    return dispatch, combine


def _expert_body(cnt_ref, disp_ref, w_ref, b_ref, g_ref, out_ref):
    e = pl.program_id(0)
    cnt = cnt_ref[0, e]
    ri = lax.broadcasted_iota(jnp.int32, (CPAD, M), 0)
    xb = jnp.where(ri < cnt, disp_ref[...], 0.0)
    ri1 = lax.broadcasted_iota(jnp.int32, (CPAD, 1), 0)
    g = jnp.where(ri1 < cnt, g_ref[:, 0:1], 0.0)
    out_ref[...] = (
        jnp.dot(xb, w_ref[0], preferred_element_type=jnp.float32) + b_ref[0]
    ) * g


_expert_in_specs = [
    pl.BlockSpec(memory_space=pltpu.SMEM),
    pl.BlockSpec((CPAD, M), lambda e: (e, 0)),
    pl.BlockSpec((1, M, M), lambda e: (e, 0, 0)),
    pl.BlockSpec((1, 1, M), lambda e: (e, 0, 0)),
    pl.BlockSpec((CPAD, GL), lambda e: (e, 0)),
]
_expert_out_specs = pl.BlockSpec((CPAD, M), lambda e: (e, 0))

_expert_mm = pl.pallas_call(
    _expert_body,
    grid=(E,),
    in_specs=_expert_in_specs,
    out_specs=_expert_out_specs,
    out_shape=jax.ShapeDtypeStruct((EC, M), jnp.float32),
)


def kernel(features, wg, W, b):
    B, S, _ = features.shape
    x = features.reshape(T, M)
    wg8 = jnp.pad(wg, ((0, 0), (0, EP - E)))
    dispatch, combine = _sc_kernels()
    gm, idx, counts, laux = _gating(x, wg8)
    disp, gslot = dispatch(x, gm, idx.reshape(T))
    eo = _expert_mm(counts, disp, W, b.reshape(E, 1, M), gslot)
    comb = combine(eo, idx.reshape(T))
    return comb.reshape(B, S, M), laux[0, 0]


# speedup vs baseline: 1.2657x; 1.1631x over previous
import functools
import math

import jax
import jax.numpy as jnp
from jax import lax
from jax.experimental import pallas as pl
from jax.experimental.pallas import tpu as pltpu
from jax.experimental.pallas import tpu_sc as plsc

T = 2048
M = 1024
E = 3
EP = 8
CAP = 683
CPAD = 688
EC = E * CPAD
DUMP = CAP
GL = 128
TBLK = 128
NBLK = T // TBLK


def _gating_body(x_ref, wg_ref, gm_ref, idx_ref, cnt_ref, laux_ref):
    logits = jnp.dot(x_ref[...], wg_ref[...],
                     preferred_element_type=jnp.float32)
    col = lax.broadcasted_iota(jnp.int32, (T, EP), 1)
    valid = col < E
    neg = jnp.float32(-1e30)
    logits = jnp.where(valid, logits, neg)

    mx = jnp.max(logits, axis=1, keepdims=True)
    ex = jnp.exp(logits - mx)
    ex = jnp.where(valid, ex, 0.0)
    gates = ex / jnp.sum(ex, axis=1, keepdims=True)

    gmax = jnp.max(gates, axis=1, keepdims=True)
    iseq = jnp.logical_and(gates == gmax, valid)
    e_s = jnp.min(jnp.where(iseq, col, 999), axis=1, keepdims=True)
    mask1 = jnp.where(col == e_s, 1.0, 0.0)

    loc = mask1
    s = 1
    while s < T:
        loc = loc + jnp.concatenate(
            [jnp.zeros((s, EP), loc.dtype), loc[:-s]], axis=0)
        s *= 2
    loc = loc - mask1

    keep = mask1 * jnp.where(loc < CAP, 1.0, 0.0)
    gm = jnp.sum(gates * keep, axis=1, keepdims=True)
    c_s = jnp.sum(loc * keep, axis=1, keepdims=True)
    kept = jnp.sum(keep, axis=1, keepdims=True)
    slot = e_s.astype(jnp.float32) * CPAD + c_s
    gm_ref[...] = jnp.broadcast_to(gm, (T, GL))
    idx_ref[...] = (kept * slot + (1.0 - kept) * DUMP).astype(jnp.int32)

    tot = jnp.sum(mask1, axis=0, keepdims=True)
    cnt_ref[...] = jnp.minimum(tot, float(CAP)).astype(jnp.int32)
    me = jnp.sum(gates, axis=0, keepdims=True) / T
    ce = tot / T
    laux_ref[...] = jnp.sum(me * ce, axis=1, keepdims=True) * E


_gating_in_specs = [
    pl.BlockSpec((T, M), lambda: (0, 0)),
    pl.BlockSpec((M, EP), lambda: (0, 0)),
]
_gating_out_specs = [
    pl.BlockSpec((T, GL), lambda: (0, 0)),
    pl.BlockSpec((T, 1), lambda: (0, 0)),
    pl.BlockSpec((1, EP), lambda: (0, 0)),
    pl.BlockSpec((1, 1), lambda: (0, 0)),
]
_gating_out_shape = [
    jax.ShapeDtypeStruct((T, GL), jnp.float32),
    jax.ShapeDtypeStruct((T, 1), jnp.int32),
    jax.ShapeDtypeStruct((1, EP), jnp.int32),
    jax.ShapeDtypeStruct((1, 1), jnp.float32),
]
_gating_scratch = []

_gating = pl.pallas_call(
    _gating_body,
    grid=(),
    in_specs=_gating_in_specs,
    out_specs=_gating_out_specs,
    out_shape=_gating_out_shape,
)


_NC = 2
_NS = 16
_NW = _NC * _NS
_ROWS_PER_W = T // _NW


def _dispatch_body(x_hbm, gm_hbm, idx_hbm, disp_hbm, gslot_hbm,
                   idx_v, rows_v, gm_v, sem, sem2):
    wid = lax.axis_index("s") * _NC + lax.axis_index("c")
    base = wid * _ROWS_PER_W
    pltpu.sync_copy(idx_hbm.at[pl.ds(base, _ROWS_PER_W)], idx_v)
    pltpu.sync_copy(x_hbm.at[pl.ds(base, _ROWS_PER_W)], rows_v)
    pltpu.sync_copy(gm_hbm.at[pl.ds(base, _ROWS_PER_W)], gm_v)
    row_cp = pltpu.async_copy(rows_v, disp_hbm.at[idx_v], sem)
    gm_cp = pltpu.async_copy(gm_v, gslot_hbm.at[idx_v], sem2)
    row_cp.wait()
    gm_cp.wait()


def _combine_body(eo_hbm, idx_hbm, out_hbm, idx_v, rows_v, sem):
    wid = lax.axis_index("s") * _NC + lax.axis_index("c")
    base = wid * _ROWS_PER_W
    pltpu.sync_copy(idx_hbm.at[pl.ds(base, _ROWS_PER_W)], idx_v)
    pltpu.async_copy(eo_hbm.at[idx_v], rows_v, sem).wait()
    pltpu.sync_copy(rows_v, out_hbm.at[pl.ds(base, _ROWS_PER_W)])


@functools.lru_cache(maxsize=None)
def _sc_kernels():
    mesh = plsc.VectorSubcoreMesh(
        core_axis_name="c", subcore_axis_name="s",
        num_cores=_NC, num_subcores=_NS,
    )
    dispatch = pl.kernel(
        _dispatch_body,
        out_type=[
            jax.ShapeDtypeStruct((EC, M), jnp.float32),
            jax.ShapeDtypeStruct((EC, GL), jnp.float32),
        ],
        mesh=mesh,
        scratch_types=[
            pltpu.VMEM((_ROWS_PER_W,), jnp.int32),
            pltpu.VMEM((_ROWS_PER_W, M), jnp.float32),
            pltpu.VMEM((_ROWS_PER_W, GL), jnp.float32),
            pltpu.SemaphoreType.DMA,
            pltpu.SemaphoreType.DMA,
        ],
    )
    combine = pl.kernel(
        _combine_body,
        out_type=jax.ShapeDtypeStruct((T, M), jnp.float32),
        mesh=mesh,
        scratch_types=[
            pltpu.VMEM((_ROWS_PER_W,), jnp.int32),
            pltpu.VMEM((_ROWS_PER_W, M), jnp.float32),
            pltpu.SemaphoreType.DMA,
        ],
    )
    return dispatch, combine


def _expert_body(cnt_ref, disp_ref, w_ref, b_ref, g_ref, out_ref):
    e = pl.program_id(0)
    cnt = cnt_ref[0, e]
    ri = lax.broadcasted_iota(jnp.int32, (CPAD, M), 0)
    xb = jnp.where(ri < cnt, disp_ref[...], 0.0)
    ri1 = lax.broadcasted_iota(jnp.int32, (CPAD, 1), 0)
    g = jnp.where(ri1 < cnt, g_ref[:, 0:1], 0.0)
    out_ref[...] = (
        jnp.dot(xb, w_ref[0], preferred_element_type=jnp.float32) + b_ref[0]
    ) * g


_expert_in_specs = [
    pl.BlockSpec(memory_space=pltpu.SMEM),
    pl.BlockSpec((CPAD, M), lambda e: (e, 0)),
    pl.BlockSpec((1, M, M), lambda e: (e, 0, 0)),
    pl.BlockSpec((1, 1, M), lambda e: (e, 0, 0)),
    pl.BlockSpec((CPAD, GL), lambda e: (e, 0)),
]
_expert_out_specs = pl.BlockSpec((CPAD, M), lambda e: (e, 0))

_expert_mm = pl.pallas_call(
    _expert_body,
    grid=(E,),
    in_specs=_expert_in_specs,
    out_specs=_expert_out_specs,
    out_shape=jax.ShapeDtypeStruct((EC, M), jnp.float32),
)


def kernel(features, wg, W, b):
    B, S, _ = features.shape
    x = features.reshape(T, M)
    wg8 = jnp.pad(wg, ((0, 0), (0, EP - E)))
    dispatch, combine = _sc_kernels()
    gm, idx, counts, laux = _gating(x, wg8)
    disp, gslot = dispatch(x, gm, idx.reshape(T))
    eo = _expert_mm(counts, disp, W, b.reshape(E, 1, M), gslot)
    comb = combine(eo, idx.reshape(T))
    return comb.reshape(B, S, M), laux[0, 0]
